# Initial kernel scaffold; baseline (speedup 1.0000x reference)
import jax
import jax.numpy as jnp
from jax.experimental import pallas as pl


def kernel(x, edge_attr, W_node, b_node, W_edge, b_edge, W_upd, b_upd, batch, subgraph_idx_batch, edge_index):
    raise NotImplementedError("write your pallas kernel here")


# speedup vs baseline: 2.5631x; 2.5631x over previous
import functools

import jax
import jax.numpy as jnp
from jax import lax
from jax.experimental import pallas as pl
from jax.experimental.pallas import tpu as pltpu
from jax.experimental.pallas import tpu_sc as plsc

N_NODES = 10000
NPAD = 10240
E = 320000
N_SUB = 512
H = 128
AW = 16

HH = H // 2

CH = 80
NCHUNK = E // CH
CPH = NCHUNK // 2 // 16
TILE_ROWS = NPAD // 16

R = 1000
NB = N_NODES // R


def _sc_segment_sums(xs, srcx, dst, ea16, zg, za):
    mesh = plsc.VectorSubcoreMesh(core_axis_name="c", subcore_axis_name="s")

    @functools.partial(
        pl.kernel,
        out_type=[
            jax.ShapeDtypeStruct((2 * NPAD, HH), jnp.float32),
            jax.ShapeDtypeStruct((2 * NPAD, AW), jnp.float32),
        ],
        mesh=mesh,
        compiler_params=pltpu.CompilerParams(use_tc_tiling_on_sc=False),
        scratch_types=[
            pltpu.VMEM((CH,), jnp.int32),
            pltpu.VMEM((CH,), jnp.int32),
            pltpu.VMEM((CH, HH), jnp.float32),
            pltpu.VMEM((CH, AW), jnp.float32),
            pltpu.VMEM((TILE_ROWS, HH), jnp.float32),
            pltpu.VMEM((TILE_ROWS, AW), jnp.float32),
            pltpu.VMEM_SHARED((NPAD, HH), jnp.float32),
            pltpu.VMEM_SHARED((NPAD, AW), jnp.float32),
            pltpu.SemaphoreType.DMA,
        ],
    )
    def sc_kernel(xs_hbm, srcx_hbm, dst_hbm, ea_hbm, zg_hbm, za_hbm,
                  g_out, a_out, srcb, dstb, rowb, eab, stg_g, stg_a,
                  gacc, aacc, sem):
        cid = lax.axis_index("c")
        sid = lax.axis_index("s")
        r0 = sid * TILE_ROWS

        pltpu.sync_copy(zg_hbm.at[pl.ds(r0, TILE_ROWS)], stg_g)
        pltpu.sync_copy(stg_g, gacc.at[pl.ds(r0, TILE_ROWS)])
        pltpu.sync_copy(za_hbm.at[pl.ds(r0, TILE_ROWS)], stg_a)
        pltpu.sync_copy(stg_a, aacc.at[pl.ds(r0, TILE_ROWS)])
        plsc.subcore_barrier()

        own0 = cid * (NCHUNK // 2) + sid * CPH
        oth0 = (1 - cid) * (NCHUNK // 2) + sid * CPH

        def do_rows(j):
            base = j * CH
            pltpu.sync_copy(srcx_hbm.at[pl.ds(cid * E + base, CH)], srcb)
            pltpu.async_copy(xs_hbm.at[srcb], rowb, sem).wait()
            pltpu.sync_copy(dst_hbm.at[pl.ds(base, CH)], dstb)
            pltpu.sync_copy(rowb, gacc.at[dstb], add=True)

        def body_own(i, carry):
            j = own0 + i
            do_rows(j)
            pltpu.sync_copy(ea_hbm.at[pl.ds(j * CH, CH)], eab)
            pltpu.sync_copy(eab, aacc.at[dstb], add=True)
            return carry

        def body_oth(i, carry):
            do_rows(oth0 + i)
            return carry

        lax.fori_loop(0, CPH, body_own, 0)
        lax.fori_loop(0, CPH, body_oth, 0)
        plsc.subcore_barrier()

        pltpu.sync_copy(gacc.at[pl.ds(r0, TILE_ROWS)], stg_g)
        pltpu.sync_copy(stg_g, g_out.at[pl.ds(cid * NPAD + r0, TILE_ROWS)])
        pltpu.sync_copy(aacc.at[pl.ds(r0, TILE_ROWS)], stg_a)
        pltpu.sync_copy(stg_a, a_out.at[pl.ds(cid * NPAD + r0, TILE_ROWS)])

    gp, ap = sc_kernel(xs, srcx, dst, ea16, zg, za)
    return gp.reshape(2, NPAD, HH), ap.reshape(2, NPAD, AW)


def _tc_body(x_ref, gp_ref, ap_ref, batch_ref, wn_ref, we_ref, wu_ref,
             bn_ref, bu_ref, out_ref, cnt_ref):
    i = pl.program_id(0)

    @pl.when(i == 0)
    def _init():
        out_ref[...] = jnp.zeros_like(out_ref)
        cnt_ref[...] = jnp.zeros_like(cnt_ref)

    xb = x_ref[...]
    a = ap_ref[0] + ap_ref[1]
    wn = wn_ref[...]
    z = (
        jnp.dot(xb[:, :HH] + gp_ref[0], wn[:HH], preferred_element_type=jnp.float32)
        + jnp.dot(xb[:, HH:] + gp_ref[1], wn[HH:], preferred_element_type=jnp.float32)
        + jnp.dot(a, we_ref[...], preferred_element_type=jnp.float32)
        + bn_ref[...]
    )
    y = jnp.maximum(jnp.dot(z, wu_ref[...], preferred_element_type=jnp.float32)
                    + bu_ref[...], 0.0)

    seg = batch_ref[0]
    onehot = (seg == lax.broadcasted_iota(jnp.int32, (N_SUB, R), 0)
              ).astype(jnp.float32)
    out_ref[...] += jnp.dot(onehot, y, preferred_element_type=jnp.float32)
    cnt_ref[...] += jnp.sum(onehot, axis=1, keepdims=True)

    @pl.when(i == NB - 1)
    def _fin():
        out_ref[...] = out_ref[...] / jnp.maximum(cnt_ref[...], 1.0)


def kernel(x, edge_attr, W_node, b_node, W_edge, b_edge, W_upd, b_upd,
           batch, subgraph_idx_batch, edge_index):
    src = edge_index[0]
    dst = edge_index[1]
    ea16 = jnp.concatenate(
        [edge_attr,
         jnp.ones((E, 1), jnp.float32),
         jnp.zeros((E, AW - 4), jnp.float32)], axis=1)
    we16 = jnp.concatenate(
        [W_edge,
         (b_node + b_edge)[None, :],
         jnp.zeros((AW - 4, H), jnp.float32)], axis=0)
    zg = jnp.zeros((NPAD, HH), jnp.float32)
    za = jnp.zeros((NPAD, AW), jnp.float32)
    xs = jnp.concatenate([x[:, :HH], x[:, HH:]], axis=0)
    srcx = jnp.concatenate([src, src + N_NODES])

    gp, ap = _sc_segment_sums(xs, srcx, dst, ea16, zg, za)

    out = pl.pallas_call(
        _tc_body,
        grid=(NB,),
        in_specs=[
            pl.BlockSpec((R, H), lambda i: (i, 0)),
            pl.BlockSpec((2, R, HH), lambda i: (0, i, 0)),
            pl.BlockSpec((2, R, AW), lambda i: (0, i, 0)),
            pl.BlockSpec((1, 1, R), lambda i: (i, 0, 0)),
            pl.BlockSpec((H, H), lambda i: (0, 0)),
            pl.BlockSpec((AW, H), lambda i: (0, 0)),
            pl.BlockSpec((H, H), lambda i: (0, 0)),
            pl.BlockSpec((1, H), lambda i: (0, 0)),
            pl.BlockSpec((1, H), lambda i: (0, 0)),
        ],
        out_specs=pl.BlockSpec((N_SUB, H), lambda i: (0, 0)),
        out_shape=jax.ShapeDtypeStruct((N_SUB, H), jnp.float32),
        scratch_shapes=[pltpu.VMEM((N_SUB, H), jnp.float32)],
    )(x, gp, ap, batch.reshape(NB, 1, R), W_node, we16, W_upd,
      b_node[None, :], b_upd[None, :])
    return out


# speedup vs baseline: 3.4164x; 1.3329x over previous
import functools

import jax
import jax.numpy as jnp
from jax import lax
from jax.experimental import pallas as pl
from jax.experimental.pallas import tpu as pltpu
from jax.experimental.pallas import tpu_sc as plsc

N_NODES = 10000
NPAD = 10240
E = 320000
N_SUB = 512
H = 128
AW = 16

HH = H // 2

CH = 80
NCHUNK = E // CH
CPH = NCHUNK // 2 // 16
TILE_ROWS = NPAD // 16

R = 1000
NB = N_NODES // R


def _sc_segment_sums(xs, srcx, dst, ea16, zg, za):
    mesh = plsc.VectorSubcoreMesh(core_axis_name="c", subcore_axis_name="s")

    @functools.partial(
        pl.kernel,
        out_type=[
            jax.ShapeDtypeStruct((2 * NPAD, HH), jnp.float32),
            jax.ShapeDtypeStruct((2 * NPAD, AW), jnp.float32),
        ],
        mesh=mesh,
        compiler_params=pltpu.CompilerParams(use_tc_tiling_on_sc=False),
        scratch_types=[
            pltpu.VMEM((CH,), jnp.int32),
            pltpu.VMEM((CH,), jnp.int32),
            pltpu.VMEM((CH,), jnp.int32),
            pltpu.VMEM((CH,), jnp.int32),
            pltpu.VMEM((CH, HH), jnp.float32),
            pltpu.VMEM((CH, HH), jnp.float32),
            pltpu.VMEM((CH, AW), jnp.float32),
            pltpu.VMEM((CH, AW), jnp.float32),
            pltpu.VMEM((TILE_ROWS, HH), jnp.float32),
            pltpu.VMEM((TILE_ROWS, AW), jnp.float32),
            pltpu.VMEM_SHARED((NPAD, HH), jnp.float32),
            pltpu.VMEM_SHARED((NPAD, AW), jnp.float32),
            pltpu.SemaphoreType.DMA,
            pltpu.SemaphoreType.DMA,
        ],
    )
    def sc_kernel(xs_hbm, srcx_hbm, dst_hbm, ea_hbm, zg_hbm, za_hbm,
                  g_out, a_out, srcb0, srcb1, dstb0, dstb1, rowb0, rowb1,
                  eab0, eab1, stg_g, stg_a, gacc, aacc, sem0, sem1):
        cid = lax.axis_index("c")
        sid = lax.axis_index("s")
        r0 = sid * TILE_ROWS
        sb = (srcb0, srcb1)
        db = (dstb0, dstb1)
        rb = (rowb0, rowb1)
        eb = (eab0, eab1)
        sems = (sem0, sem1)

        pltpu.sync_copy(zg_hbm.at[pl.ds(r0, TILE_ROWS)], stg_g)
        pltpu.sync_copy(stg_g, gacc.at[pl.ds(r0, TILE_ROWS)])
        pltpu.sync_copy(za_hbm.at[pl.ds(r0, TILE_ROWS)], stg_a)
        pltpu.sync_copy(stg_a, aacc.at[pl.ds(r0, TILE_ROWS)])
        plsc.subcore_barrier()

        own0 = cid * (NCHUNK // 2) + sid * CPH
        oth0 = (1 - cid) * (NCHUNK // 2) + sid * CPH

        def run_half(c0, own):
            def fire(j, p):
                base = j * CH
                pltpu.sync_copy(srcx_hbm.at[pl.ds(cid * E + base, CH)], sb[p])
                pltpu.sync_copy(dst_hbm.at[pl.ds(base, CH)], db[p])
                if own:
                    pltpu.sync_copy(ea_hbm.at[pl.ds(base, CH)], eb[p])
                pltpu.async_copy(xs_hbm.at[sb[p]], rb[p], sems[p])

            def drain(p):
                pltpu.make_async_copy(xs_hbm.at[sb[p]], rb[p], sems[p]).wait()
                pltpu.sync_copy(rb[p], gacc.at[db[p]], add=True)
                if own:
                    pltpu.sync_copy(eb[p], aacc.at[db[p]], add=True)

            fire(c0, 0)

            def pair(i, carry):
                j = c0 + 2 * i
                fire(j + 1, 1)
                drain(0)
                fire(j + 2, 0)
                drain(1)
                return carry

            lax.fori_loop(0, (CPH - 1) // 2, pair, 0)
            drain(0)

        run_half(own0, True)
        run_half(oth0, False)
        plsc.subcore_barrier()

        pltpu.sync_copy(gacc.at[pl.ds(r0, TILE_ROWS)], stg_g)
        pltpu.sync_copy(stg_g, g_out.at[pl.ds(cid * NPAD + r0, TILE_ROWS)])
        pltpu.sync_copy(aacc.at[pl.ds(r0, TILE_ROWS)], stg_a)
        pltpu.sync_copy(stg_a, a_out.at[pl.ds(cid * NPAD + r0, TILE_ROWS)])

    gp, ap = sc_kernel(xs, srcx, dst, ea16, zg, za)
    return gp.reshape(2, NPAD, HH), ap.reshape(2, NPAD, AW)


def _tc_body(x_ref, gp_ref, ap_ref, batch_ref, wn_ref, we_ref, wu_ref,
             bn_ref, bu_ref, out_ref, cnt_ref):
    i = pl.program_id(0)

    @pl.when(i == 0)
    def _init():
        out_ref[...] = jnp.zeros_like(out_ref)
        cnt_ref[...] = jnp.zeros_like(cnt_ref)

    xb = x_ref[...]
    a = ap_ref[0] + ap_ref[1]
    wn = wn_ref[...]
    z = (
        jnp.dot(xb[:, :HH] + gp_ref[0], wn[:HH], preferred_element_type=jnp.float32)
        + jnp.dot(xb[:, HH:] + gp_ref[1], wn[HH:], preferred_element_type=jnp.float32)
        + jnp.dot(a, we_ref[...], preferred_element_type=jnp.float32)
        + bn_ref[...]
    )
    y = jnp.maximum(jnp.dot(z, wu_ref[...], preferred_element_type=jnp.float32)
                    + bu_ref[...], 0.0)

    seg = batch_ref[0]
    onehot = (seg == lax.broadcasted_iota(jnp.int32, (N_SUB, R), 0)
              ).astype(jnp.float32)
    out_ref[...] += jnp.dot(onehot, y, preferred_element_type=jnp.float32)
    cnt_ref[...] += jnp.sum(onehot, axis=1, keepdims=True)

    @pl.when(i == NB - 1)
    def _fin():
        out_ref[...] = out_ref[...] / jnp.maximum(cnt_ref[...], 1.0)


def kernel(x, edge_attr, W_node, b_node, W_edge, b_edge, W_upd, b_upd,
           batch, subgraph_idx_batch, edge_index):
    src = edge_index[0]
    dst = edge_index[1]
    ea16 = jnp.concatenate(
        [edge_attr,
         jnp.ones((E, 1), jnp.float32),
         jnp.zeros((E, AW - 4), jnp.float32)], axis=1)
    we16 = jnp.concatenate(
        [W_edge,
         (b_node + b_edge)[None, :],
         jnp.zeros((AW - 4, H), jnp.float32)], axis=0)
    zg = jnp.zeros((NPAD, HH), jnp.float32)
    za = jnp.zeros((NPAD, AW), jnp.float32)
    xs = jnp.concatenate([x[:, :HH], x[:, HH:]], axis=0)
    srcx = jnp.concatenate([src, src + N_NODES])

    gp, ap = _sc_segment_sums(xs, srcx, dst, ea16, zg, za)

    out = pl.pallas_call(
        _tc_body,
        grid=(NB,),
        in_specs=[
            pl.BlockSpec((R, H), lambda i: (i, 0)),
            pl.BlockSpec((2, R, HH), lambda i: (0, i, 0)),
            pl.BlockSpec((2, R, AW), lambda i: (0, i, 0)),
            pl.BlockSpec((1, 1, R), lambda i: (i, 0, 0)),
            pl.BlockSpec((H, H), lambda i: (0, 0)),
            pl.BlockSpec((AW, H), lambda i: (0, 0)),
            pl.BlockSpec((H, H), lambda i: (0, 0)),
            pl.BlockSpec((1, H), lambda i: (0, 0)),
            pl.BlockSpec((1, H), lambda i: (0, 0)),
        ],
        out_specs=pl.BlockSpec((N_SUB, H), lambda i: (0, 0)),
        out_shape=jax.ShapeDtypeStruct((N_SUB, H), jnp.float32),
        scratch_shapes=[pltpu.VMEM((N_SUB, H), jnp.float32)],
    )(x, gp, ap, batch.reshape(NB, 1, R), W_node, we16, W_upd,
      b_node[None, :], b_upd[None, :])
    return out


# speedup vs baseline: 4.6751x; 1.3684x over previous
import functools

import jax
import jax.numpy as jnp
from jax import lax
from jax.experimental import pallas as pl
from jax.experimental.pallas import tpu as pltpu
from jax.experimental.pallas import tpu_sc as plsc

N_NODES = 10000
NPAD = 10240
E = 320000
N_SUB = 512
H = 128
AW = 16

HH = H // 2

CH = 80
NCHUNK = E // CH
CPH = NCHUNK // 2 // 16
TILE_ROWS = NPAD // 16

R = 1000
NB = N_NODES // R


def _sc_segment_sums(xs, srcx, dst, ea16, zg, za):
    mesh = plsc.VectorSubcoreMesh(core_axis_name="c", subcore_axis_name="s")

    @functools.partial(
        pl.kernel,
        out_type=[
            jax.ShapeDtypeStruct((2 * NPAD, HH), jnp.float32),
            jax.ShapeDtypeStruct((2 * NPAD, AW), jnp.float32),
        ],
        mesh=mesh,
        compiler_params=pltpu.CompilerParams(use_tc_tiling_on_sc=False),
        scratch_types=[
            pltpu.VMEM((CPH, CH), jnp.int32),
            pltpu.VMEM((CPH, CH), jnp.int32),
            pltpu.VMEM((CH, HH), jnp.float32),
            pltpu.VMEM((CH, HH), jnp.float32),
            pltpu.VMEM((CH, AW), jnp.float32),
            pltpu.VMEM((CH, AW), jnp.float32),
            pltpu.VMEM((TILE_ROWS // 4, HH), jnp.float32),
            pltpu.VMEM((TILE_ROWS // 4, AW), jnp.float32),
            pltpu.VMEM_SHARED((NPAD, HH), jnp.float32),
            pltpu.VMEM_SHARED((NPAD, AW), jnp.float32),
            pltpu.SemaphoreType.DMA,
            pltpu.SemaphoreType.DMA,
        ],
    )
    def sc_kernel(xs_hbm, srcx_hbm, dst_hbm, ea_hbm, zg_hbm, za_hbm,
                  g_out, a_out, srcblk, dstblk, rowb0, rowb1,
                  eab0, eab1, stg_g, stg_a, gacc, aacc, sem0, sem1):
        cid = lax.axis_index("c")
        sid = lax.axis_index("s")
        r0 = sid * TILE_ROWS
        rb = (rowb0, rowb1)
        eb = (eab0, eab1)
        sems = (sem0, sem1)

        QR = TILE_ROWS // 4
        for q in range(4):
            pltpu.sync_copy(zg_hbm.at[pl.ds(r0 + q * QR, QR)], stg_g)
            pltpu.sync_copy(stg_g, gacc.at[pl.ds(r0 + q * QR, QR)])
            pltpu.sync_copy(za_hbm.at[pl.ds(r0 + q * QR, QR)], stg_a)
            pltpu.sync_copy(stg_a, aacc.at[pl.ds(r0 + q * QR, QR)])
        plsc.subcore_barrier()

        own0 = cid * (NCHUNK // 2) + sid * CPH
        oth0 = (1 - cid) * (NCHUNK // 2) + sid * CPH

        def run_half(c0, own):
            pltpu.sync_copy(srcx_hbm.at[cid].at[pl.ds(c0, CPH)], srcblk)
            pltpu.sync_copy(dst_hbm.at[pl.ds(c0, CPH)], dstblk)

            def fire(i, p):
                if own:
                    pltpu.sync_copy(ea_hbm.at[c0 + i], eb[p])
                pltpu.async_copy(xs_hbm.at[srcblk.at[i]], rb[p], sems[p])

            def drain(i, p):
                pltpu.make_async_copy(
                    xs_hbm.at[srcblk.at[i]], rb[p], sems[p]).wait()
                pltpu.sync_copy(rb[p], gacc.at[dstblk.at[i]], add=True)
                if own:
                    pltpu.sync_copy(eb[p], aacc.at[dstblk.at[i]], add=True)

            fire(0, 0)

            def pair(k, carry):
                i = 2 * k
                fire(i + 1, 1)
                drain(i, 0)
                fire(i + 2, 0)
                drain(i + 1, 1)
                return carry

            lax.fori_loop(0, (CPH - 1) // 2, pair, 0)
            drain(CPH - 1, 0)

        run_half(own0, True)
        run_half(oth0, False)
        plsc.subcore_barrier()

        for q in range(4):
            pltpu.sync_copy(gacc.at[pl.ds(r0 + q * QR, QR)], stg_g)
            pltpu.sync_copy(
                stg_g, g_out.at[pl.ds(cid * NPAD + r0 + q * QR, QR)])
            pltpu.sync_copy(aacc.at[pl.ds(r0 + q * QR, QR)], stg_a)
            pltpu.sync_copy(
                stg_a, a_out.at[pl.ds(cid * NPAD + r0 + q * QR, QR)])

    gp, ap = sc_kernel(
        xs,
        srcx.reshape(2, NCHUNK, CH),
        dst.reshape(NCHUNK, CH),
        ea16.reshape(NCHUNK, CH, AW),
        zg, za)
    return gp.reshape(2, NPAD, HH), ap.reshape(2, NPAD, AW)


def _tc_body(x_ref, gp_ref, ap_ref, batch_ref, wn_ref, we_ref, wu_ref,
             bn_ref, bu_ref, out_ref, cnt_ref):
    i = pl.program_id(0)

    @pl.when(i == 0)
    def _init():
        out_ref[...] = jnp.zeros_like(out_ref)
        cnt_ref[...] = jnp.zeros_like(cnt_ref)

    xb = x_ref[...]
    a = ap_ref[0] + ap_ref[1]
    wn = wn_ref[...]
    z = (
        jnp.dot(xb[:, :HH] + gp_ref[0], wn[:HH], preferred_element_type=jnp.float32)
        + jnp.dot(xb[:, HH:] + gp_ref[1], wn[HH:], preferred_element_type=jnp.float32)
        + jnp.dot(a, we_ref[...], preferred_element_type=jnp.float32)
        + bn_ref[...]
    )
    y = jnp.maximum(jnp.dot(z, wu_ref[...], preferred_element_type=jnp.float32)
                    + bu_ref[...], 0.0)

    seg = batch_ref[0]
    onehot = (seg == lax.broadcasted_iota(jnp.int32, (N_SUB, R), 0)
              ).astype(jnp.float32)
    out_ref[...] += jnp.dot(onehot, y, preferred_element_type=jnp.float32)
    cnt_ref[...] += jnp.sum(onehot, axis=1, keepdims=True)

    @pl.when(i == NB - 1)
    def _fin():
        out_ref[...] = out_ref[...] / jnp.maximum(cnt_ref[...], 1.0)


def kernel(x, edge_attr, W_node, b_node, W_edge, b_edge, W_upd, b_upd,
           batch, subgraph_idx_batch, edge_index):
    src = edge_index[0]
    dst = edge_index[1]
    ea16 = jnp.concatenate(
        [edge_attr,
         jnp.ones((E, 1), jnp.float32),
         jnp.zeros((E, AW - 4), jnp.float32)], axis=1)
    we16 = jnp.concatenate(
        [W_edge,
         (b_node + b_edge)[None, :],
         jnp.zeros((AW - 4, H), jnp.float32)], axis=0)
    zg = jnp.zeros((NPAD, HH), jnp.float32)
    za = jnp.zeros((NPAD, AW), jnp.float32)
    xs = jnp.concatenate([x[:, :HH], x[:, HH:]], axis=0)
    srcx = jnp.concatenate([src, src + N_NODES])

    gp, ap = _sc_segment_sums(xs, srcx, dst, ea16, zg, za)

    out = pl.pallas_call(
        _tc_body,
        grid=(NB,),
        in_specs=[
            pl.BlockSpec((R, H), lambda i: (i, 0)),
            pl.BlockSpec((2, R, HH), lambda i: (0, i, 0)),
            pl.BlockSpec((2, R, AW), lambda i: (0, i, 0)),
            pl.BlockSpec((1, 1, R), lambda i: (i, 0, 0)),
            pl.BlockSpec((H, H), lambda i: (0, 0)),
            pl.BlockSpec((AW, H), lambda i: (0, 0)),
            pl.BlockSpec((H, H), lambda i: (0, 0)),
            pl.BlockSpec((1, H), lambda i: (0, 0)),
            pl.BlockSpec((1, H), lambda i: (0, 0)),
        ],
        out_specs=pl.BlockSpec((N_SUB, H), lambda i: (0, 0)),
        out_shape=jax.ShapeDtypeStruct((N_SUB, H), jnp.float32),
        scratch_shapes=[pltpu.VMEM((N_SUB, H), jnp.float32)],
    )(x, gp, ap, batch.reshape(NB, 1, R), W_node, we16, W_upd,
      b_node[None, :], b_upd[None, :])
    return out


# speedup vs baseline: 4.8835x; 1.0446x over previous
import functools

import jax
import jax.numpy as jnp
from jax import lax
from jax.experimental import pallas as pl
from jax.experimental.pallas import tpu as pltpu
from jax.experimental.pallas import tpu_sc as plsc

N_NODES = 10000
NPAD = 10240
E = 320000
N_SUB = 512
H = 128
AW = 8

HH = H // 2

CH = 80
NCHUNK = E // CH
CPH = NCHUNK // 2 // 16
TILE_ROWS = NPAD // 16

R = 1000
NB = N_NODES // R


def _sc_segment_sums(xs, src, dst, ea16, zg, za):
    mesh = plsc.VectorSubcoreMesh(core_axis_name="c", subcore_axis_name="s")

    @functools.partial(
        pl.kernel,
        out_type=[
            jax.ShapeDtypeStruct((2 * NPAD, HH), jnp.float32),
            jax.ShapeDtypeStruct((2 * NPAD, AW), jnp.float32),
        ],
        mesh=mesh,
        compiler_params=pltpu.CompilerParams(use_tc_tiling_on_sc=False),
        scratch_types=[
            pltpu.VMEM((CPH, CH), jnp.int32),
            pltpu.VMEM((CPH, CH), jnp.int32),
            pltpu.VMEM((CH, HH), jnp.float32),
            pltpu.VMEM((CH, HH), jnp.float32),
            pltpu.VMEM((CH, AW), jnp.float32),
            pltpu.VMEM((CH, AW), jnp.float32),
            pltpu.VMEM((TILE_ROWS // 4, HH), jnp.float32),
            pltpu.VMEM((TILE_ROWS // 4, AW), jnp.float32),
            pltpu.VMEM_SHARED((NPAD, HH), jnp.float32),
            pltpu.VMEM_SHARED((NPAD, AW), jnp.float32),
            pltpu.SemaphoreType.DMA,
            pltpu.SemaphoreType.DMA,
        ],
    )
    def sc_kernel(xs_hbm, src_hbm, dst_hbm, ea_hbm, zg_hbm, za_hbm,
                  g_out, a_out, srcblk, dstblk, rowb0, rowb1,
                  eab0, eab1, stg_g, stg_a, gacc, aacc, sem0, sem1):
        cid = lax.axis_index("c")
        sid = lax.axis_index("s")
        r0 = sid * TILE_ROWS
        rb = (rowb0, rowb1)
        eb = (eab0, eab1)
        sems = (sem0, sem1)

        QR = TILE_ROWS // 4
        for q in range(4):
            pltpu.sync_copy(zg_hbm.at[pl.ds(r0 + q * QR, QR)], stg_g)
            pltpu.sync_copy(stg_g, gacc.at[pl.ds(r0 + q * QR, QR)])
            pltpu.sync_copy(za_hbm.at[pl.ds(r0 + q * QR, QR)], stg_a)
            pltpu.sync_copy(stg_a, aacc.at[pl.ds(r0 + q * QR, QR)])
        plsc.subcore_barrier()

        own0 = cid * (NCHUNK // 2) + sid * CPH
        oth0 = (1 - cid) * (NCHUNK // 2) + sid * CPH

        def run_half(c0, own):
            pltpu.sync_copy(src_hbm.at[pl.ds(c0, CPH)], srcblk)
            pltpu.sync_copy(dst_hbm.at[pl.ds(c0, CPH)], dstblk)
            xh = xs_hbm.at[cid]

            def fire(i, p):
                if own:
                    pltpu.sync_copy(ea_hbm.at[c0 + i], eb[p])
                pltpu.async_copy(xh.at[srcblk.at[i]], rb[p], sems[p])

            def drain(i, p):
                pltpu.make_async_copy(
                    xh.at[srcblk.at[i]], rb[p], sems[p]).wait()
                pltpu.sync_copy(rb[p], gacc.at[dstblk.at[i]], add=True)
                if own:
                    pltpu.sync_copy(eb[p], aacc.at[dstblk.at[i]], add=True)

            fire(0, 0)

            def pair(k, carry):
                i = 2 * k
                fire(i + 1, 1)
                drain(i, 0)
                fire(i + 2, 0)
                drain(i + 1, 1)
                return carry

            lax.fori_loop(0, (CPH - 1) // 2, pair, 0)
            drain(CPH - 1, 0)

        run_half(own0, True)
        run_half(oth0, False)
        plsc.subcore_barrier()

        for q in range(4):
            pltpu.sync_copy(gacc.at[pl.ds(r0 + q * QR, QR)], stg_g)
            pltpu.sync_copy(
                stg_g, g_out.at[pl.ds(cid * NPAD + r0 + q * QR, QR)])
            pltpu.sync_copy(aacc.at[pl.ds(r0 + q * QR, QR)], stg_a)
            pltpu.sync_copy(
                stg_a, a_out.at[pl.ds(cid * NPAD + r0 + q * QR, QR)])

    gp, ap = sc_kernel(
        xs,
        src.reshape(NCHUNK, CH),
        dst.reshape(NCHUNK, CH),
        ea16.reshape(NCHUNK, CH, AW),
        zg, za)
    return gp.reshape(2, NPAD, HH), ap.reshape(2, NPAD, AW)


def _tc_body(x_ref, gp_ref, ap_ref, batch_ref, wn_ref, we_ref, wu_ref,
             bn_ref, bu_ref, out_ref, cnt_ref):
    i = pl.program_id(0)

    @pl.when(i == 0)
    def _init():
        out_ref[...] = jnp.zeros_like(out_ref)
        cnt_ref[...] = jnp.zeros_like(cnt_ref)

    xb = x_ref[...]
    a = ap_ref[0] + ap_ref[1]
    wn = wn_ref[...]
    z = (
        jnp.dot(xb[:, :HH] + gp_ref[0], wn[:HH], preferred_element_type=jnp.float32)
        + jnp.dot(xb[:, HH:] + gp_ref[1], wn[HH:], preferred_element_type=jnp.float32)
        + jnp.dot(a, we_ref[...], preferred_element_type=jnp.float32)
        + bn_ref[...]
    )
    y = jnp.maximum(jnp.dot(z, wu_ref[...], preferred_element_type=jnp.float32)
                    + bu_ref[...], 0.0)

    seg = batch_ref[0]
    onehot = (seg == lax.broadcasted_iota(jnp.int32, (N_SUB, R), 0)
              ).astype(jnp.float32)
    out_ref[...] += jnp.dot(onehot, y, preferred_element_type=jnp.float32)
    cnt_ref[...] += jnp.sum(onehot, axis=1, keepdims=True)

    @pl.when(i == NB - 1)
    def _fin():
        out_ref[...] = out_ref[...] / jnp.maximum(cnt_ref[...], 1.0)


def kernel(x, edge_attr, W_node, b_node, W_edge, b_edge, W_upd, b_upd,
           batch, subgraph_idx_batch, edge_index):
    src = edge_index[0]
    dst = edge_index[1]
    ea16 = jnp.concatenate(
        [edge_attr,
         jnp.ones((E, 1), jnp.float32),
         jnp.zeros((E, AW - 4), jnp.float32)], axis=1)
    we16 = jnp.concatenate(
        [W_edge,
         (b_node + b_edge)[None, :],
         jnp.zeros((AW - 4, H), jnp.float32)], axis=0)
    zg = jnp.zeros((NPAD, HH), jnp.float32)
    za = jnp.zeros((NPAD, AW), jnp.float32)
    xs = jnp.stack([x[:, :HH], x[:, HH:]])

    gp, ap = _sc_segment_sums(xs, src, dst, ea16, zg, za)

    out = pl.pallas_call(
        _tc_body,
        grid=(NB,),
        in_specs=[
            pl.BlockSpec((R, H), lambda i: (i, 0)),
            pl.BlockSpec((2, R, HH), lambda i: (0, i, 0)),
            pl.BlockSpec((2, R, AW), lambda i: (0, i, 0)),
            pl.BlockSpec((1, 1, R), lambda i: (i, 0, 0)),
            pl.BlockSpec((H, H), lambda i: (0, 0)),
            pl.BlockSpec((AW, H), lambda i: (0, 0)),
            pl.BlockSpec((H, H), lambda i: (0, 0)),
            pl.BlockSpec((1, H), lambda i: (0, 0)),
            pl.BlockSpec((1, H), lambda i: (0, 0)),
        ],
        out_specs=pl.BlockSpec((N_SUB, H), lambda i: (0, 0)),
        out_shape=jax.ShapeDtypeStruct((N_SUB, H), jnp.float32),
        scratch_shapes=[pltpu.VMEM((N_SUB, H), jnp.float32)],
    )(x, gp, ap, batch.reshape(NB, 1, R), W_node, we16, W_upd,
      b_node[None, :], b_upd[None, :])
    return out
